# baseline (speedup 1.0000x reference)
import functools

import jax
import jax.numpy as jnp
from jax import lax
from jax.experimental import pallas as pl
from jax.experimental.pallas import tpu as pltpu
from jax.experimental.pallas import tpu_sc as plsc

_V = 1001
_VP = 1002
_D = 300
_DP = 320
_B = 4096 * 50
_NC = 2
_NS = 16
_NW = _NC * _NS
_BPW = _B // _NW
_CHUNK = 64
_NCHUNK = _BPW // _CHUNK
_CB = _CHUNK * _D
_NPAIR = _VP // 2

_mesh = plsc.VectorSubcoreMesh(core_axis_name="c", subcore_axis_name="s")


_IDXR = _B // 128


@functools.partial(
    pl.kernel,
    mesh=_mesh,
    compiler_params=pltpu.CompilerParams(use_tc_tiling_on_sc=False),
    out_type=jax.ShapeDtypeStruct((_B * _D,), jnp.float32),
    scratch_types=[
        pltpu.HBM((_VP, _DP), jnp.float32),
        pltpu.VMEM((_BPW // 128, 128), jnp.int32),
        pltpu.VMEM((600,), jnp.float32),
        pltpu.VMEM((2, _DP), jnp.float32),
        pltpu.VMEM((_CHUNK, _DP), jnp.float32),
        pltpu.VMEM((_CHUNK, _DP), jnp.float32),
        pltpu.VMEM((_CB,), jnp.float32),
        pltpu.VMEM((_CB,), jnp.float32),
        pltpu.SemaphoreType.DMA,
        pltpu.SemaphoreType.DMA,
        pltpu.SemaphoreType.DMA,
        pltpu.SemaphoreType.DMA,
    ],
)
def _emb_gather(idx_hbm, wflat_hbm, out_hbm, tbl, idx_v, rowp_v, padp_v,
                pad0, pad1, cmp0, cmp1, gs0, gs1, ws0, ws1):
    sid = lax.axis_index("s")
    wid = sid * _NC + lax.axis_index("c")
    base = wid * _BPW * _D

    def stage_pair(g, carry):
        pltpu.sync_copy(wflat_hbm.at[pl.ds(g * 2 * _D, 2 * _D)], rowp_v)
        for r in range(2):
            for k in range(18):
                padp_v[r, pl.ds(16 * k, 16)] = rowp_v[pl.ds(r * _D + 16 * k, 16)]
            padp_v[r, pl.ds(_D - 16, 16)] = rowp_v[pl.ds(r * _D + _D - 16, 16)]
        pltpu.sync_copy(padp_v, tbl.at[pl.ds(2 * g, 2)])
        return carry

    lax.fori_loop(0, (_NPAIR - sid + _NS - 1) // _NS,
                  lambda i, c: stage_pair(sid + i * _NS, c), 0)
    plsc.subcore_barrier()

    pltpu.sync_copy(idx_hbm.at[pl.ds(wid * (_BPW // 128), _BPW // 128)], idx_v)

    def gather_idx(j):
        return idx_v.at[j // 2, pl.ds(64 * (j % 2), _CHUNK)]

    pltpu.async_copy(tbl.at[gather_idx(0)], pad0, gs0)

    def compact(pad_v, cmp_v):
        @plsc.parallel_loop(0, _CHUNK, unroll=4)
        def _(r):
            for k in range(18):
                cmp_v[pl.ds(_D * r + 16 * k, 16)] = pad_v[r, pl.ds(16 * k, 16)]
            cmp_v[pl.ds(_D * r + _D - 16, 16)] = pad_v[r, pl.ds(_D - 16, 16)]

    def out_flat(j):
        return out_hbm.at[pl.ds(base + j * _CB, _CB)]

    def pair(t, carry):
        j0 = 2 * t
        pltpu.async_copy(tbl.at[gather_idx(j0 + 1)], pad1, gs1)
        pltpu.make_async_copy(tbl.at[gather_idx(j0)], pad0, gs0).wait()

        @pl.when(t >= 1)
        def _():
            pltpu.make_async_copy(cmp0, out_flat(j0 - 2), ws0).wait()

        compact(pad0, cmp0)
        pltpu.async_copy(cmp0, out_flat(j0), ws0)

        @pl.when(t < _NCHUNK // 2 - 1)
        def _():
            pltpu.async_copy(tbl.at[gather_idx(j0 + 2)], pad0, gs0)

        pltpu.make_async_copy(tbl.at[gather_idx(j0 + 1)], pad1, gs1).wait()

        @pl.when(t >= 1)
        def _():
            pltpu.make_async_copy(cmp1, out_flat(j0 - 1), ws1).wait()

        compact(pad1, cmp1)
        pltpu.async_copy(cmp1, out_flat(j0 + 1), ws1)
        return carry

    lax.fori_loop(0, _NCHUNK // 2, pair, 0)
    pltpu.make_async_copy(cmp0, out_flat(_NCHUNK - 2), ws0).wait()
    pltpu.make_async_copy(cmp1, out_flat(_NCHUNK - 1), ws1).wait()


def kernel(sentence, W):
    idx = sentence.reshape(_IDXR, 128)
    wflat = jnp.concatenate([W.reshape(_V * _D), jnp.zeros((_D,), W.dtype)])
    out = _emb_gather(idx, wflat)
    return out.reshape(sentence.shape[0], sentence.shape[1], _D)


# speedup vs baseline: 1.0015x; 1.0015x over previous
import functools

import jax
import jax.numpy as jnp
from jax import lax
from jax.experimental import pallas as pl
from jax.experimental.pallas import tpu as pltpu
from jax.experimental.pallas import tpu_sc as plsc

_V = 1001
_VP = 1002
_D = 300
_DP = 320
_B = 4096 * 50
_NC = 2
_NS = 16
_NW = _NC * _NS
_BPW = _B // _NW
_CHUNK = 64
_NCHUNK = _BPW // _CHUNK
_CB = _CHUNK * _D
_NPAIR = _VP // 2

_mesh = plsc.VectorSubcoreMesh(core_axis_name="c", subcore_axis_name="s")


_IDXR = _B // 128


@functools.partial(
    pl.kernel,
    mesh=_mesh,
    compiler_params=pltpu.CompilerParams(use_tc_tiling_on_sc=False),
    out_type=jax.ShapeDtypeStruct((_B * _D,), jnp.float32),
    scratch_types=[
        pltpu.HBM((_VP, _DP), jnp.float32),
        pltpu.VMEM((_BPW // 128, 128), jnp.int32),
        pltpu.VMEM((600,), jnp.float32),
        pltpu.VMEM((2, _DP), jnp.float32),
        pltpu.VMEM((_CHUNK, _DP), jnp.float32),
        pltpu.VMEM((_CHUNK, _DP), jnp.float32),
        pltpu.VMEM((_CB,), jnp.float32),
        pltpu.VMEM((_CB,), jnp.float32),
        pltpu.SemaphoreType.DMA,
        pltpu.SemaphoreType.DMA,
        pltpu.SemaphoreType.DMA,
        pltpu.SemaphoreType.DMA,
    ],
)
def _emb_gather(idx_hbm, wflat_hbm, out_hbm, tbl, idx_v, rowp_v, padp_v,
                pad0, pad1, cmp0, cmp1, gs0, gs1, ws0, ws1):
    sid = lax.axis_index("s")
    wid = sid * _NC + lax.axis_index("c")
    base = wid * _BPW * _D

    def stage_pair(g, carry):
        pltpu.sync_copy(wflat_hbm.at[pl.ds(g * 2 * _D, 2 * _D)], rowp_v)
        for r in range(2):
            for k in range(18):
                padp_v[r, pl.ds(16 * k, 16)] = rowp_v[pl.ds(r * _D + 16 * k, 16)]
            padp_v[r, pl.ds(_D - 16, 16)] = rowp_v[pl.ds(r * _D + _D - 16, 16)]
        pltpu.sync_copy(padp_v, tbl.at[pl.ds(2 * g, 2)])
        return carry

    lax.fori_loop(0, (_NPAIR - sid + _NS - 1) // _NS,
                  lambda i, c: stage_pair(sid + i * _NS, c), 0)
    plsc.subcore_barrier()

    pltpu.sync_copy(idx_hbm.at[pl.ds(wid * (_BPW // 128), _BPW // 128)], idx_v)

    def gather_idx(j):
        return idx_v.at[j // 2, pl.ds(64 * (j % 2), _CHUNK)]

    pltpu.async_copy(tbl.at[gather_idx(0)], pad0, gs0)

    def compact(pad_v, cmp_v):
        @plsc.parallel_loop(0, _CHUNK, unroll=4)
        def _(r):
            for k in range(18):
                cmp_v[pl.ds(_D * r + 16 * k, 16)] = pad_v[r, pl.ds(16 * k, 16)]
            cmp_v[pl.ds(_D * r + _D - 16, 16)] = pad_v[r, pl.ds(_D - 16, 16)]

    def out_flat(j):
        return out_hbm.at[pl.ds(base + j * _CB, _CB)]

    def pair(t, carry):
        j0 = 2 * t
        pltpu.async_copy(tbl.at[gather_idx(j0 + 1)], pad1, gs1)
        pltpu.make_async_copy(tbl.at[gather_idx(j0)], pad0, gs0).wait()

        @pl.when(t >= 1)
        def _():
            pltpu.make_async_copy(cmp0, out_flat(j0 - 2), ws0).wait()

        compact(pad0, cmp0)
        pltpu.async_copy(cmp0, out_flat(j0), ws0)

        @pl.when(t < _NCHUNK // 2 - 1)
        def _():
            pltpu.async_copy(tbl.at[gather_idx(j0 + 2)], pad0, gs0)

        pltpu.make_async_copy(tbl.at[gather_idx(j0 + 1)], pad1, gs1).wait()

        @pl.when(t >= 1)
        def _():
            pltpu.make_async_copy(cmp1, out_flat(j0 - 1), ws1).wait()

        compact(pad1, cmp1)
        pltpu.async_copy(cmp1, out_flat(j0 + 1), ws1)
        return carry

    lax.fori_loop(0, _NCHUNK // 2, pair, 0)
    pltpu.make_async_copy(cmp0, out_flat(_NCHUNK - 2), ws0).wait()
    pltpu.make_async_copy(cmp1, out_flat(_NCHUNK - 1), ws1).wait()


def kernel(sentence, W):
    idx = jnp.bitwise_and(sentence.reshape(_IDXR, 128), jnp.int32(0xFFFF))
    wflat = jnp.concatenate([W.reshape(_V * _D), jnp.zeros((_D,), W.dtype)])
    out = _emb_gather(idx, wflat)
    return out.reshape(sentence.shape[0], sentence.shape[1], _D)
